# Initial kernel scaffold; baseline (speedup 1.0000x reference)
import jax
import jax.numpy as jnp
from jax.experimental import pallas as pl


def kernel(input, wg, w1, b1, w2, b2):
    raise NotImplementedError("write your pallas kernel here")


# speedup vs baseline: 2.1608x; 2.1608x over previous
import functools

import jax
import jax.numpy as jnp
from jax.experimental import pallas as pl


def _weff_kernel(w1_ref, w2_ref, b1_ref, b2_ref, weff_ref, beff_ref):
    hstep = pl.program_id(1)
    w2b = w2_ref[0]
    ones = jnp.ones((1, w2b.shape[1]), jnp.float32)
    s = jax.lax.dot_general(ones, w2b, (((1,), (1,)), ((), ())),
                            preferred_element_type=jnp.float32)
    w1b = w1_ref[0]
    part = jax.lax.dot_general(s, w1b, (((1,), (1,)), ((), ())),
                               preferred_element_type=jnp.float32)
    bpart = jnp.sum(b1_ref[0] * s)

    @pl.when(hstep == 0)
    def _init():
        weff_ref[0] = part
        beff_ref[0] = jnp.broadcast_to(bpart + jnp.sum(b2_ref[0]), (1, 128))

    @pl.when(hstep > 0)
    def _acc():
        weff_ref[0] = weff_ref[0] + part
        beff_ref[0] = beff_ref[0] + bpart


def _route_kernel(x_ref, wg_ref, weff_ref, beff_ref, out_ref, *, cap):
    x = x_ref[...]
    t, e = x.shape[0], wg_ref.shape[1]
    logits = jnp.dot(x, wg_ref[...], preferred_element_type=jnp.float32)
    scores = jax.lax.dot_general(x, weff_ref[...], (((1,), (1,)), ((), ())),
                                 preferred_element_type=jnp.float32)
    scores = scores + beff_ref[...]

    lane = jax.lax.broadcasted_iota(jnp.int32, (t, e), 1)
    mx1 = jnp.max(logits, axis=-1, keepdims=True)
    idx1 = jnp.min(jnp.where(logits == mx1, lane, e), axis=-1, keepdims=True)
    m1 = (lane == idx1).astype(jnp.float32)
    logits2 = jnp.where(m1 > 0, -jnp.inf, logits)
    mx2 = jnp.max(logits2, axis=-1, keepdims=True)
    idx2 = jnp.min(jnp.where(logits2 == mx2, lane, e), axis=-1, keepdims=True)
    m2 = (lane == idx2).astype(jnp.float32)

    eg = jnp.exp(logits - mx1)
    gates = eg / jnp.sum(eg, axis=-1, keepdims=True)
    g1 = jnp.sum(gates * m1, axis=-1, keepdims=True)
    g2 = jnp.sum(gates * m2, axis=-1, keepdims=True)
    den = g1 + g2 + 1e-9
    g1n = g1 / den
    g2n = g2 / den

    row = jax.lax.broadcasted_iota(jnp.int32, (t, t), 0)
    col = jax.lax.broadcasted_iota(jnp.int32, (t, t), 1)
    tri = (row > col).astype(jnp.float32)
    loc1 = jnp.dot(tri, m1, preferred_element_type=jnp.float32)
    count1 = jnp.sum(m1, axis=0, keepdims=True)
    loc2 = jnp.dot(tri, m2, preferred_element_type=jnp.float32) + count1
    m1k = m1 * (loc1 < cap).astype(jnp.float32)
    m2k = m2 * (loc2 < cap).astype(jnp.float32)

    s1 = jnp.sum(scores * m1k, axis=-1, keepdims=True)
    s2 = jnp.sum(scores * m2k, axis=-1, keepdims=True)
    osum = g1n * s1 + g2n * s2

    mo = jnp.max(osum, axis=0, keepdims=True)
    z = osum - mo
    lse = jnp.log(jnp.sum(jnp.exp(z), axis=0, keepdims=True))
    out_ref[...] = z - lse


def kernel(input, wg, w1, b1, w2, b2):
    b, s, d = input.shape
    t = b * s
    e = wg.shape[1]
    h = w1.shape[2]
    cap = (2 * t) // e

    xf = input.reshape(t, d)
    b1r = b1.reshape(e, 1, h)
    b2r = b2.reshape(e, 1, d)

    hb = 1024
    weff3, beff3 = pl.pallas_call(
        _weff_kernel,
        grid=(e, h // hb),
        in_specs=[
            pl.BlockSpec((1, d, hb), lambda i, j: (i, 0, j)),
            pl.BlockSpec((1, hb, d), lambda i, j: (i, j, 0)),
            pl.BlockSpec((1, 1, hb), lambda i, j: (i, 0, j)),
            pl.BlockSpec((1, 1, d), lambda i, j: (i, 0, 0)),
        ],
        out_specs=[
            pl.BlockSpec((1, 1, d), lambda i, j: (i, 0, 0)),
            pl.BlockSpec((1, 1, 128), lambda i, j: (i, 0, 0)),
        ],
        out_shape=[
            jax.ShapeDtypeStruct((e, 1, d), jnp.float32),
            jax.ShapeDtypeStruct((e, 1, 128), jnp.float32),
        ],
    )(w1, w2, b1r, b2r)
    weff = weff3.reshape(e, d)
    beff = beff3[:, 0, 0].reshape(1, e)

    out = pl.pallas_call(
        functools.partial(_route_kernel, cap=float(cap)),
        out_shape=jax.ShapeDtypeStruct((t, 1), jnp.float32),
    )(xf, wg, weff, beff)
    return out.reshape(b, s)


# speedup vs baseline: 2.3291x; 1.0779x over previous
import functools

import jax
import jax.numpy as jnp
from jax.experimental import pallas as pl
from jax.experimental.pallas import tpu as pltpu


def _w2sum_kernel(w2_ref, b1_ref, b2_ref, w2sum_ref, beff_ref):
    hstep = pl.program_id(1)
    w2b = w2_ref[0]
    ones = jnp.ones((1, w2b.shape[1]), jnp.float32)
    s = jax.lax.dot_general(ones, w2b, (((1,), (1,)), ((), ())),
                            preferred_element_type=jnp.float32)
    w2sum_ref[0] = s
    bpart = jnp.sum(b1_ref[0] * s)

    @pl.when(hstep == 0)
    def _init():
        beff_ref[0] = jnp.broadcast_to(bpart + jnp.sum(b2_ref[0]), (1, 128))

    @pl.when(hstep > 0)
    def _acc():
        beff_ref[0] = beff_ref[0] + bpart


def _weff_kernel(w1_ref, w2sum_ref, weff_ref):
    w1b = w1_ref[0]
    s = w2sum_ref[0]
    weff_ref[0] = jax.lax.dot_general(s, w1b, (((1,), (1,)), ((), ())),
                                      preferred_element_type=jnp.float32)


def _cumsum_lanes(a, n):
    col = jax.lax.broadcasted_iota(jnp.int32, a.shape, 1)
    acc = a
    sh = 1
    while sh < n:
        rolled = pltpu.roll(acc, sh, axis=1)
        acc = acc + jnp.where(col >= sh, rolled, 0.0)
        sh *= 2
    return acc


def _route_kernel(x_ref, w_ref, beff_ref, out_ref, *, cap):
    x = x_ref[...]
    t = x.shape[0]
    e = w_ref.shape[0] // 2
    ls = jax.lax.dot_general(w_ref[...], x, (((1,), (1,)), ((), ())),
                             preferred_element_type=jnp.float32)
    lg = ls[:e]
    sc = ls[e:] + beff_ref[...]

    eidx = jax.lax.broadcasted_iota(jnp.int32, (e, t), 0)
    mx1 = jnp.max(lg, axis=0, keepdims=True)
    idx1 = jnp.min(jnp.where(lg == mx1, eidx, e), axis=0, keepdims=True)
    m1 = (eidx == idx1).astype(jnp.float32)
    lg2 = jnp.where(m1 > 0, -jnp.inf, lg)
    mx2 = jnp.max(lg2, axis=0, keepdims=True)
    idx2 = jnp.min(jnp.where(lg2 == mx2, eidx, e), axis=0, keepdims=True)
    m2 = (eidx == idx2).astype(jnp.float32)

    eg = jnp.exp(lg - mx1)
    gates = eg / jnp.sum(eg, axis=0, keepdims=True)
    g1 = jnp.sum(gates * m1, axis=0, keepdims=True)
    g2 = jnp.sum(gates * m2, axis=0, keepdims=True)
    den = g1 + g2 + 1e-9
    g1n = g1 / den
    g2n = g2 / den

    loc1 = _cumsum_lanes(m1, t) - m1
    count1 = jnp.sum(m1, axis=1, keepdims=True)
    loc2 = _cumsum_lanes(m2, t) - m2 + count1
    m1k = m1 * (loc1 < cap).astype(jnp.float32)
    m2k = m2 * (loc2 < cap).astype(jnp.float32)

    comb = m1k * g1n + m2k * g2n
    osum = jnp.sum(sc * comb, axis=0, keepdims=True)

    mo = jnp.max(osum, axis=1, keepdims=True)
    z = osum - mo
    lse = jnp.log(jnp.sum(jnp.exp(z), axis=1, keepdims=True))
    out_ref[...] = z - lse


def kernel(input, wg, w1, b1, w2, b2):
    b, s, d = input.shape
    t = b * s
    e = wg.shape[1]
    h = w1.shape[2]
    cap = (2 * t) // e

    xf = input.reshape(t, d)
    b1r = b1.reshape(e, 1, h)
    b2r = b2.reshape(e, 1, d)

    hb = 2048
    w2sum, beff3 = pl.pallas_call(
        _w2sum_kernel,
        grid=(e, h // hb),
        in_specs=[
            pl.BlockSpec((1, hb, d), lambda i, j: (i, j, 0)),
            pl.BlockSpec((1, 1, hb), lambda i, j: (i, 0, j)),
            pl.BlockSpec((1, 1, d), lambda i, j: (i, 0, 0)),
        ],
        out_specs=[
            pl.BlockSpec((1, 1, hb), lambda i, j: (i, 0, j)),
            pl.BlockSpec((1, 1, 128), lambda i, j: (i, 0, 0)),
        ],
        out_shape=[
            jax.ShapeDtypeStruct((e, 1, h), jnp.float32),
            jax.ShapeDtypeStruct((e, 1, 128), jnp.float32),
        ],
    )(w2, b1r, b2r)

    db = 512
    weff3 = pl.pallas_call(
        _weff_kernel,
        grid=(e, d // db),
        in_specs=[
            pl.BlockSpec((1, db, h), lambda i, j: (i, j, 0)),
            pl.BlockSpec((1, 1, h), lambda i, j: (i, 0, 0)),
        ],
        out_specs=pl.BlockSpec((1, 1, db), lambda i, j: (i, 0, j)),
        out_shape=jax.ShapeDtypeStruct((e, 1, d), jnp.float32),
    )(w1, w2sum)

    wcat = jnp.concatenate([wg.T, weff3.reshape(e, d)], axis=0)
    beff = beff3[:, :, 0]

    out = pl.pallas_call(
        functools.partial(_route_kernel, cap=float(cap)),
        out_shape=jax.ShapeDtypeStruct((1, t), jnp.float32),
    )(xf, wcat, beff)
    return out.reshape(b, s)
